# baseline (speedup 1.0000x reference)
import functools

import jax
import jax.numpy as jnp
from jax import lax
from jax.experimental import pallas as pl
from jax.experimental.pallas import tpu as pltpu
from jax.experimental.pallas import tpu_sc as plsc

BATCH = 16384
EMBED_DIM = 32
NUM_CORES = 2
NUM_SUBCORES = 16
LANES = 16
NUM_WORKERS = NUM_CORES * NUM_SUBCORES
B_PER_W = BATCH // NUM_WORKERS


def _dot_kernel(uid_hbm, iid_hbm, ut_hbm, it_hbm, out_hbm,
                uidx_v, iidx_v, urows_v, irows_v, out_v, sem_u, sem_i):
    wid = lax.axis_index("s") * NUM_CORES + lax.axis_index("c")
    base = pl.multiple_of(wid * B_PER_W, B_PER_W)

    pltpu.sync_copy(uid_hbm.at[pl.ds(base, B_PER_W)], uidx_v)
    pltpu.sync_copy(iid_hbm.at[pl.ds(base, B_PER_W)], iidx_v)
    cp_u = pltpu.async_copy(ut_hbm.at[uidx_v], urows_v, sem_u)
    cp_i = pltpu.async_copy(it_hbm.at[iidx_v], irows_v, sem_i)
    cp_u.wait()
    cp_i.wait()

    lane_iota = lax.iota(jnp.int32, LANES)

    def block_body(blk, _):
        row_idx = blk * LANES + lane_iota
        acc = jnp.zeros((LANES,), jnp.float32)
        for d in range(EMBED_DIM):
            col_idx = jnp.full((LANES,), d, jnp.int32)
            u = plsc.load_gather(urows_v, [row_idx, col_idx])
            v = plsc.load_gather(irows_v, [row_idx, col_idx])
            acc = acc + u * v
        start = pl.multiple_of(blk * LANES, LANES)
        out_v[pl.ds(start, LANES)] = acc
        return _

    lax.fori_loop(0, B_PER_W // LANES, block_body, None)

    pltpu.sync_copy(out_v, out_hbm.at[pl.ds(base, B_PER_W)])


@jax.jit
def _run(user_ids, item_ids, user_table, item_table):
    mesh = plsc.VectorSubcoreMesh(core_axis_name="c", subcore_axis_name="s")
    return pl.kernel(
        _dot_kernel,
        mesh=mesh,
        out_type=jax.ShapeDtypeStruct((BATCH,), jnp.float32),
        scratch_types=[
            pltpu.VMEM((B_PER_W,), jnp.int32),
            pltpu.VMEM((B_PER_W,), jnp.int32),
            pltpu.VMEM((B_PER_W, EMBED_DIM), jnp.float32),
            pltpu.VMEM((B_PER_W, EMBED_DIM), jnp.float32),
            pltpu.VMEM((B_PER_W,), jnp.float32),
            pltpu.SemaphoreType.DMA,
            pltpu.SemaphoreType.DMA,
        ],
        compiler_params=pltpu.CompilerParams(
            needs_layout_passes=False, use_tc_tiling_on_sc=False),
    )(user_ids, item_ids, user_table, item_table)


def kernel(user_ids, item_ids, user_table, item_table):
    return _run(user_ids.astype(jnp.int32), item_ids.astype(jnp.int32),
                user_table, item_table)


# speedup vs baseline: 1.4907x; 1.4907x over previous
import functools

import jax
import jax.numpy as jnp
from jax import lax
from jax.experimental import pallas as pl
from jax.experimental.pallas import tpu as pltpu
from jax.experimental.pallas import tpu_sc as plsc

BATCH = 16384
EMBED_DIM = 32
NUM_CORES = 2
NUM_SUBCORES = 16
LANES = 16
NUM_WORKERS = NUM_CORES * NUM_SUBCORES
B_PER_W = BATCH // NUM_WORKERS
NUM_GROUPS = B_PER_W // LANES


CHUNK = 256
GROUPS_PER_CHUNK = CHUNK // LANES


def _dot_kernel(uid_hbm, iid_hbm, ut_hbm, it_hbm, out_hbm,
                uidx_v, iidx_v, urows_v, irows_v, out_v, sem_u, sem_i):
    wid = lax.axis_index("s") * NUM_CORES + lax.axis_index("c")
    base = pl.multiple_of(wid * B_PER_W, B_PER_W)

    pltpu.sync_copy(uid_hbm.at[pl.ds(base, B_PER_W)], uidx_v)
    pltpu.sync_copy(iid_hbm.at[pl.ds(base, B_PER_W)], iidx_v)

    lane_iota = lax.iota(jnp.int32, LANES)

    def chunk_body(ck, _):
        cbase = pl.multiple_of(ck * CHUNK, CHUNK)

        def issue_body(g, _):
            gstart = pl.multiple_of(cbase + g * LANES, LANES)
            u16 = uidx_v[pl.ds(gstart, LANES)]
            i16 = iidx_v[pl.ds(gstart, LANES)]
            for l in range(LANES):
                j = g * LANES + l
                pltpu.async_copy(ut_hbm.at[u16[l]], urows_v.at[j], sem_u)
                pltpu.async_copy(it_hbm.at[i16[l]], irows_v.at[j], sem_i)
            return _

        lax.fori_loop(0, GROUPS_PER_CHUNK, issue_body, None)

        pltpu.make_async_copy(ut_hbm.at[pl.ds(0, CHUNK)], urows_v,
                              sem_u).wait()
        pltpu.make_async_copy(ut_hbm.at[pl.ds(0, CHUNK)], irows_v,
                              sem_i).wait()

        def block_body(blk, _):
            row_idx = blk * LANES + lane_iota
            acc = jnp.zeros((LANES,), jnp.float32)
            for d in range(EMBED_DIM):
                col_idx = jnp.full((LANES,), d, jnp.int32)
                u = plsc.load_gather(urows_v, [row_idx, col_idx])
                v = plsc.load_gather(irows_v, [row_idx, col_idx])
                acc = acc + u * v
            start = pl.multiple_of(cbase + blk * LANES, LANES)
            out_v[pl.ds(start, LANES)] = acc
            return _

        lax.fori_loop(0, GROUPS_PER_CHUNK, block_body, None)
        return _

    lax.fori_loop(0, B_PER_W // CHUNK, chunk_body, None)

    pltpu.sync_copy(out_v, out_hbm.at[pl.ds(base, B_PER_W)])


@jax.jit
def _run(user_ids, item_ids, user_table, item_table):
    mesh = plsc.VectorSubcoreMesh(core_axis_name="c", subcore_axis_name="s")
    return pl.kernel(
        _dot_kernel,
        mesh=mesh,
        out_type=jax.ShapeDtypeStruct((BATCH,), jnp.float32),
        scratch_types=[
            pltpu.VMEM((B_PER_W,), jnp.int32),
            pltpu.VMEM((B_PER_W,), jnp.int32),
            pltpu.VMEM((CHUNK, EMBED_DIM), jnp.float32),
            pltpu.VMEM((CHUNK, EMBED_DIM), jnp.float32),
            pltpu.VMEM((B_PER_W,), jnp.float32),
            pltpu.SemaphoreType.DMA,
            pltpu.SemaphoreType.DMA,
        ],
        compiler_params=pltpu.CompilerParams(needs_layout_passes=False),
    )(user_ids, item_ids, user_table, item_table)


def kernel(user_ids, item_ids, user_table, item_table):
    return _run(user_ids.astype(jnp.int32), item_ids.astype(jnp.int32),
                user_table, item_table)
